# Initial kernel scaffold; baseline (speedup 1.0000x reference)
import jax
import jax.numpy as jnp
from jax.experimental import pallas as pl


def kernel(feat, edge_index, W_in, b_in, W_gat, b_gat, att_src, att_dst, W_ih, W_hh, W_out, b_out):
    raise NotImplementedError("write your pallas kernel here")


# speedup vs baseline: 16.0124x; 16.0124x over previous
import functools

import jax
import jax.numpy as jnp
from jax import lax
from jax.experimental import pallas as pl
from jax.experimental.pallas import tpu as pltpu
from jax.experimental.pallas import tpu_sc as plsc

NC = 2
NS = 16
NW = NC * NS
L = 16
CHUNK = 128


def rows_per_tile(n):
    r = (n + NS - 1) // NS
    return r + (-r % 8)


def half_rows(n):
    r = -(-n // (2 * NS))
    r += -r % 8
    return NS * r


def _dense_prep(feat_ref, win_ref, bin_ref, wgat_ref, asrc_ref, adst_ref,
                x_ref, sc_ref, *, hops):
    x = jnp.dot(feat_ref[...], win_ref[...]) + bin_ref[...]
    x_ref[...] = x
    cols = []
    for i in range(hops):
        h = jnp.dot(x, wgat_ref[i])
        cols.append(jnp.sum(h * asrc_ref[i, :][None, :], axis=1, keepdims=True))
        cols.append(jnp.sum(h * adst_ref[i, :][None, :], axis=1, keepdims=True))
    pad = jnp.zeros((x.shape[0], 8 - 2 * hops), jnp.float32)
    sc_ref[...] = jnp.concatenate(cols + [pad], axis=1)


def _dense_finish(x_ref, num_ref, den_ref, wgat_ref, bgat_ref, wih_ref,
                  whh_ref, wout_ref, bout_ref, out_ref, *, hops):
    x = x_ref[...]
    h = x
    c = jnp.zeros_like(x)
    dn = (((1,), (1,)), ((), ()))
    for i in range(hops):
        num = num_ref[0, i] + num_ref[1, i]
        den = den_ref[0, i, :, :1] + den_ref[1, i, :, :1]
        agg = num / (den + 1e-16)
        g = jnp.tanh(jnp.dot(agg, wgat_ref[i]) + bgat_ref[i, :][None, :])
        gates = (lax.dot_general(g, wih_ref[i], dn)
                 + lax.dot_general(h, whh_ref[i], dn))
        H = x.shape[1]
        ig = jax.nn.sigmoid(gates[:, :H])
        fg = jax.nn.sigmoid(gates[:, H:2 * H])
        gg = jnp.tanh(gates[:, 2 * H:3 * H])
        og = jax.nn.sigmoid(gates[:, 3 * H:])
        c = fg * c + ig * gg
        h = og * jnp.tanh(c)
    out_ref[...] = jnp.dot(h, wout_ref[...]) + bout_ref[...]


def _sc_edge_body(x_hbm, scores_hbm, esrc_hbm, edst_hbm, num_hbm, den_hbm,
                  asn_l, adn_l, srci, dsti, dsth, rows, payn, payd,
                  num_acc, den_acc, *, n, hops, n_chunks):
    cid = lax.axis_index("c")
    sid = lax.axis_index("s")
    wid = sid * NC + cid

    max_cnt = (n_chunks + NW - 1) // NW
    max_cnt += -max_cnt % 8
    start = wid * max_cnt
    cnt = jnp.maximum(0, jnp.minimum(max_cnt, n_chunks - start))

    nh = half_rows(n)
    rows_t = nh // NS

    pltpu.sync_copy(esrc_hbm.at[pl.ds(start, max_cnt)], srci)
    pltpu.sync_copy(edst_hbm.at[pl.ds(start, max_cnt)], dsti)

    idx16 = lax.iota(jnp.int32, L)
    zcol = jnp.zeros((L,), jnp.int32)
    zv = jnp.zeros((L,), jnp.float32)

    for hop in range(hops):
        pltpu.sync_copy(scores_hbm.at[2 * hop], asn_l)
        pltpu.sync_copy(scores_hbm.at[2 * hop + 1], adn_l)
        for half in range(2):
            base = half * nh
            def zero_body(j, _):
                for kk in range(CHUNK // L):
                    rows[j, pl.ds(kk * L, L)] = zv
                payd[j, :] = zv
                return 0
            lax.fori_loop(0, CHUNK, zero_body, 0)
            done = 0
            while done < rows_t:
                z = min(CHUNK, rows_t - done)
                off = sid * rows_t + done
                pltpu.sync_copy(rows.at[pl.ds(0, z)], num_acc.at[pl.ds(off, z)])
                pltpu.sync_copy(payd.at[pl.ds(0, z)], den_acc.at[pl.ds(off, z)])
                done += z
            plsc.subcore_barrier()

            def chunk_body(k, _):
                @pl.when(k < cnt)
                def _():
                    def ex_body(i, _):
                        sv = srci[k, pl.ds(i * L, L)]
                        dv = dsti[k, pl.ds(i * L, L)]
                        e = (plsc.load_gather(asn_l, [sv])
                             + plsc.load_gather(adn_l, [dv]))
                        e = jnp.where(e >= 0.0, e, 0.2 * e)
                        ex = jnp.exp(e)
                        plsc.store_scatter(payd, [idx16 + i * L, zcol], ex)
                        dvh = dv - base
                        dvh = jnp.where((dvh >= 0) & (dvh < nh), dvh, nh)
                        dsth[0, pl.ds(i * L, L)] = dvh
                        return 0
                    lax.fori_loop(0, CHUNK // L, ex_body, 0)

                    pltpu.sync_copy(x_hbm.at[srci.at[k]], rows)

                    def scale_body(j, _):
                        s = payd[j, :][0]
                        for kk in range(CHUNK // L):
                            payn[j, pl.ds(kk * L, L)] = (
                                rows[j, pl.ds(kk * L, L)] * s)
                        return 0
                    lax.fori_loop(0, CHUNK, scale_body, 0)

                    pltpu.sync_copy(payn, num_acc.at[dsth.at[0]], add=True)
                    pltpu.sync_copy(payd, den_acc.at[dsth.at[0]], add=True)
                return 0
            lax.fori_loop(0, max_cnt, chunk_body, 0)

            plsc.subcore_barrier()

            off = sid * rows_t
            pltpu.sync_copy(num_acc.at[pl.ds(off, rows_t)],
                            num_hbm.at[cid, hop, pl.ds(base + off, rows_t)])
            pltpu.sync_copy(den_acc.at[pl.ds(off, rows_t)],
                            den_hbm.at[cid, hop, pl.ds(base + off, rows_t)])
            plsc.subcore_barrier()


def kernel(feat, edge_index, W_in, b_in, W_gat, b_gat, att_src, att_dst,
           W_ih, W_hh, W_out, b_out):
    n, dfeat = feat.shape
    hh = W_in.shape[1]
    hops = W_gat.shape[0]
    e = edge_index.shape[1]
    out_d = W_out.shape[1]
    n_chunks = e // CHUNK
    assert e % CHUNK == 0 and n % 8 == 0
    n_acc = NS * rows_per_tile(n)
    nh = half_rows(n)
    n_out = 2 * nh

    x, scores = pl.pallas_call(
        functools.partial(_dense_prep, hops=hops),
        out_shape=[
            jax.ShapeDtypeStruct((n, hh), jnp.float32),
            jax.ShapeDtypeStruct((n, 8), jnp.float32),
        ],
    )(feat, W_in, b_in.reshape(1, hh), W_gat, att_src, att_dst)

    scores_t = jnp.pad(scores.T, ((0, 0), (0, n_acc - n)))
    max_cnt = (n_chunks + NW - 1) // NW
    max_cnt += -max_cnt % 8
    n_pad = NW * max_cnt - n_chunks
    esrc = jnp.pad(edge_index[0].reshape(n_chunks, CHUNK), ((0, n_pad), (0, 0)))
    edst = jnp.pad(edge_index[1].reshape(n_chunks, CHUNK), ((0, n_pad), (0, 0)))

    mesh = plsc.VectorSubcoreMesh(core_axis_name="c", subcore_axis_name="s")
    num, den = pl.kernel(
        functools.partial(_sc_edge_body, n=n, hops=hops, n_chunks=n_chunks),
        out_type=[
            jax.ShapeDtypeStruct((NC, hops, n_out, hh), jnp.float32),
            jax.ShapeDtypeStruct((NC, hops, n_out, L), jnp.float32),
        ],
        mesh=mesh,
        compiler_params=pltpu.CompilerParams(needs_layout_passes=False,
                                             use_tc_tiling_on_sc=False),
        scratch_types=[
            pltpu.VMEM((n_acc,), jnp.float32),
            pltpu.VMEM((n_acc,), jnp.float32),
            pltpu.VMEM((max_cnt, CHUNK), jnp.int32),
            pltpu.VMEM((max_cnt, CHUNK), jnp.int32),
            pltpu.VMEM((1, CHUNK), jnp.int32),
            pltpu.VMEM((CHUNK, hh), jnp.float32),
            pltpu.VMEM((CHUNK, hh), jnp.float32),
            pltpu.VMEM((CHUNK, L), jnp.float32),
            pltpu.VMEM_SHARED((nh + 8, hh), jnp.float32),
            pltpu.VMEM_SHARED((nh + 8, L), jnp.float32),
        ],
    )(x, scores_t, esrc, edst)

    out = pl.pallas_call(
        functools.partial(_dense_finish, hops=hops),
        out_shape=jax.ShapeDtypeStruct((n, out_d), jnp.float32),
        grid=(10,),
        in_specs=[
            pl.BlockSpec((n // 10, hh), lambda i: (i, 0)),
            pl.BlockSpec((NC, hops, n // 10, hh), lambda i: (0, 0, i, 0)),
            pl.BlockSpec((NC, hops, n // 10, L), lambda i: (0, 0, i, 0)),
            pl.BlockSpec(W_gat.shape, lambda i: (0, 0, 0)),
            pl.BlockSpec((hops, hh), lambda i: (0, 0)),
            pl.BlockSpec(W_ih.shape, lambda i: (0, 0, 0)),
            pl.BlockSpec(W_hh.shape, lambda i: (0, 0, 0)),
            pl.BlockSpec(W_out.shape, lambda i: (0, 0)),
            pl.BlockSpec((1, out_d), lambda i: (0, 0)),
        ],
        out_specs=pl.BlockSpec((n // 10, out_d), lambda i: (i, 0)),
    )(x, num, den, W_gat, b_gat, W_ih, W_hh, W_out, b_out.reshape(1, out_d))
    return out
